# baseline (speedup 1.0000x reference)
import functools
import math

import jax
import jax.numpy as jnp
from jax import lax
from jax.experimental import pallas as pl
from jax.experimental.pallas import tpu as pltpu
from jax.experimental.pallas import tpu_sc as plsc

NUM_CORES = 2
NUM_SUBCORES = 16
NW = NUM_CORES * NUM_SUBCORES
CHUNK = 800
LANES = 16
NBUF = 2


@functools.cache
def _build(B, D):
    b_per_w = B // NW
    n_chunks = b_per_w // CHUNK
    n_outer = n_chunks // NBUF
    scale = math.sqrt(D)
    mesh = plsc.VectorSubcoreMesh(core_axis_name="c", subcore_axis_name="s")

    @functools.partial(
        pl.kernel,
        mesh=mesh,
        out_type=jax.ShapeDtypeStruct((NW, n_chunks, CHUNK, D), jnp.float32),
        scratch_types=[
            pltpu.VMEM((n_chunks, CHUNK), jnp.int32),
            pltpu.VMEM((NBUF, CHUNK, D), jnp.float32),
            pltpu.SemaphoreType.DMA((NBUF,)),
            pltpu.SemaphoreType.DMA((NBUF,)),
        ],
        compiler_params=pltpu.CompilerParams(use_tc_tiling_on_sc=False),
    )
    def k(x_hbm, lut_hbm, out_hbm, idx_v, rows_v, gsem, ssem):
        wid = lax.axis_index("s") * NUM_CORES + lax.axis_index("c")
        pltpu.sync_copy(x_hbm.at[wid], idx_v)

        def start_gather(ci, b):
            pltpu.async_copy(lut_hbm.at[idx_v.at[ci]], rows_v.at[b], gsem.at[b])

        def wait_gather(ci, b):
            pltpu.make_async_copy(
                lut_hbm.at[idx_v.at[ci]], rows_v.at[b], gsem.at[b]
            ).wait()

        def start_scatter(ci, b):
            pltpu.async_copy(rows_v.at[b], out_hbm.at[wid, ci], ssem.at[b])

        def wait_scatter(ci, b):
            pltpu.make_async_copy(
                rows_v.at[b], out_hbm.at[wid, ci], ssem.at[b]
            ).wait()

        def scale_buf(b):
            @plsc.parallel_loop(0, CHUNK, unroll=8)
            def _(ri):
                for j in range(D // LANES):
                    sl = (b, ri, pl.ds(j * LANES, LANES))
                    rows_v[sl] = rows_v[sl] * scale

        start_gather(0, 0)

        @pl.loop(0, n_outer)
        def _(g):
            for b in range(NBUF):
                ci = g * NBUF + b
                bo = (b + 1) % NBUF
                wait_gather(ci, b)
                scale_buf(b)
                if b == 0 and NBUF > 1:
                    @pl.when(g > 0)
                    def _():
                        wait_scatter(ci - 1, bo)
                elif NBUF > 1:
                    wait_scatter(ci - 1, bo)

                @pl.when(ci + 1 < n_chunks)
                def _():
                    start_gather(ci + 1, bo)

                start_scatter(ci, b)

        wait_scatter(n_chunks - 1, (n_chunks - 1) % NBUF)

    return k


def kernel(x, lut):
    D = lut.shape[1]
    xf = x.reshape(-1).astype(jnp.int32)
    B = xf.shape[0]
    xf = xf.reshape(NW, B // NW // CHUNK, CHUNK)
    out = _build(B, D)(xf, lut)
    return out.reshape(x.shape + (D,))


# speedup vs baseline: 1.0175x; 1.0175x over previous
import functools
import math

import jax
import jax.numpy as jnp
from jax import lax
from jax.experimental import pallas as pl
from jax.experimental.pallas import tpu as pltpu
from jax.experimental.pallas import tpu_sc as plsc

NUM_CORES = 2
NUM_SUBCORES = 16
NW = NUM_CORES * NUM_SUBCORES
CHUNK = 640
LANES = 16
NBUF = 2


@functools.cache
def _build(B, D):
    b_per_w = B // NW
    n_chunks = b_per_w // CHUNK
    n_outer = n_chunks // NBUF
    scale = math.sqrt(D)
    mesh = plsc.VectorSubcoreMesh(core_axis_name="c", subcore_axis_name="s")

    @functools.partial(
        pl.kernel,
        mesh=mesh,
        out_type=jax.ShapeDtypeStruct((NW, n_chunks, CHUNK, D), jnp.float32),
        scratch_types=[
            pltpu.VMEM((n_chunks, CHUNK), jnp.int32),
            pltpu.VMEM((NBUF, CHUNK, D), jnp.float32),
            pltpu.SemaphoreType.DMA((NBUF,)),
            pltpu.SemaphoreType.DMA((NBUF,)),
        ],
        compiler_params=pltpu.CompilerParams(use_tc_tiling_on_sc=False),
    )
    def k(x_hbm, lut_hbm, out_hbm, idx_v, rows_v, gsem, ssem):
        wid = lax.axis_index("s") * NUM_CORES + lax.axis_index("c")
        pltpu.sync_copy(x_hbm.at[wid], idx_v)

        def start_gather(ci, b):
            for kk in range(CHUNK // LANES):
                iv = idx_v[ci, pl.ds(kk * LANES, LANES)]
                pltpu.async_copy(
                    lut_hbm.at[iv],
                    rows_v.at[b, pl.ds(kk * LANES, LANES), :],
                    gsem.at[b],
                )

        def wait_gather(ci, b):
            pltpu.make_async_copy(
                lut_hbm.at[pl.ds(0, CHUNK)], rows_v.at[b], gsem.at[b]
            ).wait()

        def start_scatter(ci, b):
            pltpu.async_copy(rows_v.at[b], out_hbm.at[wid, ci], ssem.at[b])

        def wait_scatter(ci, b):
            pltpu.make_async_copy(
                rows_v.at[b], out_hbm.at[wid, ci], ssem.at[b]
            ).wait()

        def scale_buf(b):
            @plsc.parallel_loop(0, CHUNK, unroll=8)
            def _(ri):
                for j in range(D // LANES):
                    sl = (b, ri, pl.ds(j * LANES, LANES))
                    rows_v[sl] = rows_v[sl] * scale

        start_gather(0, 0)

        @pl.loop(0, n_outer)
        def _(g):
            for b in range(NBUF):
                ci = g * NBUF + b
                bo = (b + 1) % NBUF
                if b == 0:
                    @pl.when(g > 0)
                    def _():
                        wait_scatter(ci - 1, bo)
                else:
                    wait_scatter(ci - 1, bo)

                @pl.when(ci + 1 < n_chunks)
                def _():
                    start_gather(ci + 1, bo)

                wait_gather(ci, b)
                scale_buf(b)
                start_scatter(ci, b)

        wait_scatter(n_chunks - 1, (n_chunks - 1) % NBUF)

    return k


def kernel(x, lut):
    D = lut.shape[1]
    xf = x.reshape(-1).astype(jnp.int32)
    B = xf.shape[0]
    xf = xf.reshape(NW, B // NW // CHUNK, CHUNK)
    out = _build(B, D)(xf, lut)
    return out.reshape(x.shape + (D,))


# speedup vs baseline: 1.0189x; 1.0013x over previous
import functools
import math

import jax
import jax.numpy as jnp
from jax import lax
from jax.experimental import pallas as pl
from jax.experimental.pallas import tpu as pltpu
from jax.experimental.pallas import tpu_sc as plsc

NUM_CORES = 2
NUM_SUBCORES = 16
NW = NUM_CORES * NUM_SUBCORES
CHUNK = 640
LANES = 16
NBUF = 2


@functools.cache
def _build(B, D):
    b_per_w = B // NW
    n_chunks = b_per_w // CHUNK
    n_outer = n_chunks // NBUF
    scale = math.sqrt(D)
    mesh = plsc.VectorSubcoreMesh(core_axis_name="c", subcore_axis_name="s")

    @functools.partial(
        pl.kernel,
        mesh=mesh,
        out_type=jax.ShapeDtypeStruct((B, D), jnp.float32),
        scratch_types=[
            pltpu.VMEM((b_per_w,), jnp.int32),
            pltpu.VMEM((NBUF, CHUNK, D), jnp.float32),
            pltpu.SemaphoreType.DMA((NBUF,)),
            pltpu.SemaphoreType.DMA((NBUF,)),
        ],
        compiler_params=pltpu.CompilerParams(use_tc_tiling_on_sc=False),
    )
    def k(x_hbm, lut_hbm, out_hbm, idx_v, rows_v, gsem, ssem):
        wid = lax.axis_index("s") * NUM_CORES + lax.axis_index("c")
        base = wid * b_per_w
        pltpu.sync_copy(x_hbm.at[wid], idx_v)

        def start_gather(ci, b):
            for kk in range(CHUNK // LANES):
                iv = idx_v[pl.ds(ci * CHUNK + kk * LANES, LANES)]
                pltpu.async_copy(
                    lut_hbm.at[iv],
                    rows_v.at[b, pl.ds(kk * LANES, LANES), :],
                    gsem.at[b],
                )

        def wait_gather(ci, b):
            pltpu.make_async_copy(
                lut_hbm.at[pl.ds(0, CHUNK)], rows_v.at[b], gsem.at[b]
            ).wait()

        def start_scatter(ci, b):
            pltpu.async_copy(
                rows_v.at[b], out_hbm.at[pl.ds(base + ci * CHUNK, CHUNK)],
                ssem.at[b],
            )

        def wait_scatter(ci, b):
            pltpu.make_async_copy(
                rows_v.at[b], out_hbm.at[pl.ds(base + ci * CHUNK, CHUNK)],
                ssem.at[b],
            ).wait()

        def scale_buf(b):
            @plsc.parallel_loop(0, CHUNK, unroll=8)
            def _(ri):
                for j in range(D // LANES):
                    sl = (b, ri, pl.ds(j * LANES, LANES))
                    rows_v[sl] = rows_v[sl] * scale

        start_gather(0, 0)

        @pl.loop(0, n_outer)
        def _(g):
            for b in range(NBUF):
                ci = g * NBUF + b
                bo = (b + 1) % NBUF
                if b == 0:
                    @pl.when(g > 0)
                    def _():
                        wait_scatter(ci - 1, bo)
                else:
                    wait_scatter(ci - 1, bo)

                @pl.when(ci + 1 < n_chunks)
                def _():
                    start_gather(ci + 1, bo)

                wait_gather(ci, b)
                scale_buf(b)
                start_scatter(ci, b)

        wait_scatter(n_chunks - 1, (n_chunks - 1) % NBUF)

    return k


def kernel(x, lut):
    D = lut.shape[1]
    xf = x.reshape(-1).astype(jnp.int32)
    B = xf.shape[0]
    xf = xf.reshape(NW, B // NW)
    out = _build(B, D)(xf, lut)
    return out.reshape(x.shape + (D,))
